# Initial kernel scaffold; baseline (speedup 1.0000x reference)
import jax
import jax.numpy as jnp
from jax.experimental import pallas as pl


def kernel(x, edge_index, batch, conv1_Wl, conv1_bl, conv1_Wr, conv2_Wl, conv2_bl, conv2_Wr, conv3_Wl, conv3_bl, conv3_Wr, lin1_W, lin1_b, lin2_W, lin2_b):
    raise NotImplementedError("write your pallas kernel here")


# speedup vs baseline: 2.9364x; 2.9364x over previous
import functools

import jax
import jax.numpy as jnp
from jax import lax
from jax.experimental import pallas as pl
from jax.experimental.pallas import tpu as pltpu
from jax.experimental.pallas import tpu_sc as plsc

N_NODES = 10000
N_PAD = 10752
B_PAD = 10240
N_EDGES = 320000
E_PAD = 327680
NSUB = 16
N_CHUNKS = 160
CHUNK = 128
IBLK = 16
STRIPE = N_PAD // NSUB
NUM_GRAPHS = 64
MAX_NODES = 512
K = 10
HIDDEN = 256
NEG = -1e30


def _sc_mesh():
    return plsc.VectorSubcoreMesh(
        core_axis_name="c", subcore_axis_name="s", num_cores=2, num_subcores=16
    )


@functools.cache
def _make_seg_sum(dh):
    def body(h0_hbm, h1_hbm, idx_hbm, zeros_hbm, out0, out1,
             src_v, dst_v, rows_v, acc, sem):
        cid = lax.axis_index("c")
        sid = lax.axis_index("s")
        pltpu.sync_copy(zeros_hbm.at[pl.ds(sid * STRIPE, STRIPE)],
                        acc.at[pl.ds(sid * STRIPE, STRIPE)])
        plsc.subcore_barrier()

        def run(table):
            def blk(b, carry):
                pltpu.sync_copy(idx_hbm.at[0, sid, pl.ds(b * IBLK, IBLK)],
                                src_v)
                pltpu.sync_copy(idx_hbm.at[1, sid, pl.ds(b * IBLK, IBLK)],
                                dst_v)

                def step(j, carry2):
                    pltpu.async_copy(table.at[src_v.at[j]], rows_v, sem).wait()
                    pltpu.sync_copy(rows_v, acc.at[dst_v.at[j]], add=True)
                    return carry2
                lax.fori_loop(0, IBLK, step, 0)
                return carry
            lax.fori_loop(0, N_CHUNKS // IBLK, blk, 0)

        @pl.when(cid == 0)
        def _():
            run(h0_hbm)

        @pl.when(cid == 1)
        def _():
            run(h1_hbm)

        plsc.subcore_barrier()

        @pl.when(cid == 0)
        def _():
            pltpu.sync_copy(acc.at[pl.ds(sid * STRIPE, STRIPE)],
                            out0.at[pl.ds(sid * STRIPE, STRIPE)])

        @pl.when(cid == 1)
        def _():
            pltpu.sync_copy(acc.at[pl.ds(sid * STRIPE, STRIPE)],
                            out1.at[pl.ds(sid * STRIPE, STRIPE)])

    return pl.kernel(
        body,
        out_type=[jax.ShapeDtypeStruct((N_PAD, dh), jnp.float32)] * 2,
        mesh=_sc_mesh(),
        scratch_types=[
            pltpu.VMEM((IBLK, CHUNK), jnp.int32),
            pltpu.VMEM((IBLK, CHUNK), jnp.int32),
            pltpu.VMEM((CHUNK, dh), jnp.float32),
            pltpu.VMEM_SHARED((N_PAD, dh), jnp.float32),
            pltpu.SemaphoreType.DMA,
        ],
    )


@functools.cache
def _make_seg_partial():
    dh = 128
    half_chunks = N_CHUNKS // 2

    def body(h_hbm, idx_hbm, zeros_hbm, out0, out1,
             src_v, dst_v, rows_v, acc, sem):
        cid = lax.axis_index("c")
        sid = lax.axis_index("s")
        base = cid * half_chunks
        pltpu.sync_copy(zeros_hbm.at[pl.ds(sid * STRIPE, STRIPE)],
                        acc.at[pl.ds(sid * STRIPE, STRIPE)])
        plsc.subcore_barrier()

        def blk(b, carry):
            pltpu.sync_copy(idx_hbm.at[0, sid, pl.ds(base + b * IBLK, IBLK)],
                            src_v)
            pltpu.sync_copy(idx_hbm.at[1, sid, pl.ds(base + b * IBLK, IBLK)],
                            dst_v)

            def step(j, carry2):
                pltpu.async_copy(h_hbm.at[src_v.at[j]], rows_v, sem).wait()
                pltpu.sync_copy(rows_v, acc.at[dst_v.at[j]], add=True)
                return carry2
            lax.fori_loop(0, IBLK, step, 0)
            return carry
        lax.fori_loop(0, half_chunks // IBLK, blk, 0)

        plsc.subcore_barrier()

        @pl.when(cid == 0)
        def _():
            pltpu.sync_copy(acc.at[pl.ds(sid * STRIPE, STRIPE)],
                            out0.at[pl.ds(sid * STRIPE, STRIPE)])

        @pl.when(cid == 1)
        def _():
            pltpu.sync_copy(acc.at[pl.ds(sid * STRIPE, STRIPE)],
                            out1.at[pl.ds(sid * STRIPE, STRIPE)])

    return pl.kernel(
        body,
        out_type=[jax.ShapeDtypeStruct((N_PAD, dh), jnp.float32)] * 2,
        mesh=_sc_mesh(),
        scratch_types=[
            pltpu.VMEM((IBLK, CHUNK), jnp.int32),
            pltpu.VMEM((IBLK, CHUNK), jnp.int32),
            pltpu.VMEM((CHUNK, dh), jnp.float32),
            pltpu.VMEM_SHARED((N_PAD, dh), jnp.float32),
            pltpu.SemaphoreType.DMA,
        ],
    )


@functools.cache
def _make_deg_cnt():
    def body(dst_hbm, batch_hbm, ones_hbm, zeros_hbm, deg_out, cnt_out,
             idx_v, ones_v, acc, sem):
        cid = lax.axis_index("c")
        sid = lax.axis_index("s")
        pltpu.sync_copy(ones_hbm, ones_v)
        pltpu.sync_copy(zeros_hbm.at[pl.ds(sid * STRIPE, STRIPE)],
                        acc.at[pl.ds(sid * STRIPE, STRIPE)])
        plsc.subcore_barrier()

        @pl.when(cid == 0)
        def _():
            def blk(b, carry):
                pltpu.sync_copy(dst_hbm.at[sid, pl.ds(b * IBLK, IBLK)],
                                idx_v)

                def step(j, carry2):
                    pltpu.sync_copy(ones_v, acc.at[idx_v.at[j]], add=True)
                    return carry2
                lax.fori_loop(0, IBLK, step, 0)
                return carry
            lax.fori_loop(0, N_CHUNKS // IBLK, blk, 0)

        @pl.when(cid == 1)
        def _():
            pltpu.sync_copy(batch_hbm.at[sid], idx_v.at[pl.ds(0, 5)])

            def step(j, carry):
                pltpu.sync_copy(ones_v, acc.at[idx_v.at[j]], add=True)
                return carry
            lax.fori_loop(0, 5, step, 0)

        plsc.subcore_barrier()

        @pl.when(cid == 0)
        def _():
            pltpu.sync_copy(acc.at[pl.ds(sid * STRIPE, STRIPE)],
                            deg_out.at[pl.ds(sid * STRIPE, STRIPE)])

        @pl.when(jnp.logical_and(cid == 1, sid == 0))
        def _():
            pltpu.sync_copy(acc.at[pl.ds(0, CHUNK)], cnt_out)

    return pl.kernel(
        body,
        out_type=[jax.ShapeDtypeStruct((N_PAD, 128), jnp.float32),
                  jax.ShapeDtypeStruct((CHUNK, 128), jnp.float32)],
        mesh=_sc_mesh(),
        scratch_types=[
            pltpu.VMEM((IBLK, CHUNK), jnp.int32),
            pltpu.VMEM((CHUNK, 128), jnp.float32),
            pltpu.VMEM_SHARED((N_PAD, 128), jnp.float32),
            pltpu.SemaphoreType.DMA,
        ],
    )


BR = 512


@functools.cache
def _make_layer1():
    din = 128

    def body(h, p0, p1, deg, wl, bl, wr, o0, o1):
        agg = p0[...] + p1[...]
        rdeg = 1.0 / jnp.maximum(deg[...][:, :1], 1.0)
        y = (jnp.dot(agg * rdeg, wl[...], preferred_element_type=jnp.float32)
             + bl[...]
             + jnp.dot(h[...], wr[...], preferred_element_type=jnp.float32))
        y = jnp.maximum(y, 0.0)
        o0[...] = y[:, :HIDDEN // 2]
        o1[...] = y[:, HIDDEN // 2:]

    grid = N_PAD // BR
    blk = lambda c: pl.BlockSpec((BR, c), lambda i: (i, 0))
    full = lambda r, c: pl.BlockSpec((r, c), lambda i: (0, 0))
    return pl.pallas_call(
        body,
        grid=(grid,),
        in_specs=[blk(din), blk(din), blk(din),
                  pl.BlockSpec((BR, 128), lambda i: (i, 0)),
                  full(din, HIDDEN), full(1, HIDDEN), full(din, HIDDEN)],
        out_specs=[pl.BlockSpec((BR, HIDDEN // 2), lambda i: (i, 0))] * 2,
        out_shape=[jax.ShapeDtypeStruct((N_PAD, HIDDEN // 2), jnp.float32)] * 2,
    )


@functools.cache
def _make_layer(dh_in):
    din = 2 * dh_in

    def body(h0, h1, a0, a1, deg, wl, bl, wr, o0, o1):
        h = jnp.concatenate([h0[...], h1[...]], axis=1)
        agg = jnp.concatenate([a0[...], a1[...]], axis=1)
        rdeg = 1.0 / jnp.maximum(deg[...][:, :1], 1.0)
        y = (jnp.dot(agg * rdeg, wl[...], preferred_element_type=jnp.float32)
             + bl[...]
             + jnp.dot(h, wr[...], preferred_element_type=jnp.float32))
        y = jnp.maximum(y, 0.0)
        o0[...] = y[:, :HIDDEN // 2]
        o1[...] = y[:, HIDDEN // 2:]

    grid = N_PAD // BR
    half = pl.BlockSpec((BR, dh_in), lambda i: (i, 0))
    full = lambda r, c: pl.BlockSpec((r, c), lambda i: (0, 0))
    return pl.pallas_call(
        body,
        grid=(grid,),
        in_specs=[half, half, half, half,
                  pl.BlockSpec((BR, 128), lambda i: (i, 0)),
                  full(din, HIDDEN), full(1, HIDDEN), full(din, HIDDEN)],
        out_specs=[pl.BlockSpec((BR, HIDDEN // 2), lambda i: (i, 0))] * 2,
        out_shape=[jax.ShapeDtypeStruct((N_PAD, HIDDEN // 2), jnp.float32)] * 2,
    )


@functools.cache
def _make_pool_mlp():
    hh = HIDDEN // 2

    def body(h0_ref, h1_ref, cnt_ref, l1w_ref, l1b_ref, l2w_ref, l2b_ref,
             out_ref, keys_ref, pooled_ref):
        counts = cnt_ref[...][:, :1]
        r64 = lax.broadcasted_iota(jnp.int32, (NUM_GRAPHS, NUM_GRAPHS), 0)
        c64 = lax.broadcasted_iota(jnp.int32, (NUM_GRAPHS, NUM_GRAPHS), 1)
        tril = jnp.where(c64 < r64, 1.0, 0.0)
        starts = jnp.dot(tril, counts,
                         preferred_element_type=jnp.float32).astype(jnp.int32)
        cnt_i = counts.astype(jnp.int32)

        rio_col = lax.broadcasted_iota(jnp.int32, (MAX_NODES, 1), 0)
        for g in range(NUM_GRAPHS):
            s = starts[g, 0]
            krow = h1_ref[pl.ds(s, MAX_NODES), pl.ds(hh - 1, 1)]
            krow = jnp.where(rio_col < cnt_i[g, 0], krow, NEG)
            keys_ref[:, pl.ds(g, 1)] = krow

        keys = keys_ref[...]
        rio = lax.broadcasted_iota(jnp.int32, (MAX_NODES, NUM_GRAPHS), 0)
        for k in range(K):
            m = jnp.max(keys, axis=0, keepdims=True)
            cand = jnp.where(keys == m, rio, MAX_NODES * 2)
            idx = jnp.min(cand, axis=0, keepdims=True)
            for g in range(NUM_GRAPHS):
                s = starts[g, 0] + idx[0, g]
                row = jnp.concatenate(
                    [h0_ref[pl.ds(s, 1), :], h1_ref[pl.ds(s, 1), :]], axis=1)
                row = jnp.where(m[0, g] > (NEG * 0.5), row, 0.0)
                pooled_ref[pl.ds(g, 1), pl.ds(k * HIDDEN, HIDDEN)] = row
            keys = jnp.where(rio == idx, NEG, keys)

        pooled = pooled_ref[...]
        z = jnp.dot(pooled, l1w_ref[...],
                    preferred_element_type=jnp.float32) + l1b_ref[...]
        z = jnp.maximum(z, 0.0)
        logits = jnp.dot(z, l2w_ref[...],
                         preferred_element_type=jnp.float32) + l2b_ref[...]
        cmask = lax.broadcasted_iota(jnp.int32, logits.shape, 1) < 10
        lmax = jnp.max(jnp.where(cmask, logits, NEG), axis=1, keepdims=True)
        ex = jnp.where(cmask, jnp.exp(logits - lmax), 0.0)
        lse = jnp.log(jnp.sum(ex, axis=1, keepdims=True))
        out_ref[...] = logits - lmax - lse

    return pl.pallas_call(
        body,
        out_shape=jax.ShapeDtypeStruct((NUM_GRAPHS, 128), jnp.float32),
        scratch_shapes=[
            pltpu.VMEM((MAX_NODES, NUM_GRAPHS), jnp.float32),
            pltpu.VMEM((NUM_GRAPHS, K * HIDDEN), jnp.float32),
        ],
    )


def kernel(x, edge_index, batch,
           conv1_Wl, conv1_bl, conv1_Wr,
           conv2_Wl, conv2_bl, conv2_Wr,
           conv3_Wl, conv3_bl, conv3_Wr,
           lin1_W, lin1_b, lin2_W, lin2_b):
    f32 = jnp.float32
    x_pad = jnp.pad(x, ((0, N_PAD - N_NODES), (0, 0)))
    pad_e = E_PAD - N_EDGES
    dump = jnp.full((pad_e,), N_PAD - 1, jnp.int32)
    src_p = jnp.concatenate([edge_index[0], dump])
    dst_p = jnp.concatenate([edge_index[1], dump])
    idx = jnp.stack([src_p, dst_p]).reshape(2, NSUB, N_CHUNKS, CHUNK)
    dst_rs = dst_p.reshape(NSUB, N_CHUNKS, CHUNK)
    batch_rs = jnp.concatenate(
        [batch, jnp.full((B_PAD - N_NODES,), NUM_GRAPHS, jnp.int32)]
    ).reshape(NSUB, 5, CHUNK)
    ones128 = jnp.ones((CHUNK, 128), f32)
    zeros128 = jnp.zeros((N_PAD, 128), f32)

    w1l, w1r = conv1_Wl.T, conv1_Wr.T
    w2l, w2r = conv2_Wl.T, conv2_Wr.T
    w3l, w3r = conv3_Wl.T, conv3_Wr.T
    b1 = conv1_bl.reshape(1, HIDDEN)
    b2 = conv2_bl.reshape(1, HIDDEN)
    b3 = conv3_bl.reshape(1, HIDDEN)
    l1w = lin1_W.T
    l1b = lin1_b.reshape(1, HIDDEN)
    l2w = jnp.pad(lin2_W.T, ((0, 0), (0, 128 - lin2_W.shape[0])))
    l2b = jnp.pad(lin2_b, (0, 128 - lin2_b.shape[0])).reshape(1, 128)

    deg16, cnt128 = _make_deg_cnt()(dst_rs, batch_rs, ones128, zeros128)
    cnt64 = cnt128[:NUM_GRAPHS]

    p0, p1 = _make_seg_partial()(x_pad, idx, zeros128)
    h0, h1 = _make_layer1()(x_pad, p0, p1, deg16, w1l, b1, w1r)

    a0, a1 = _make_seg_sum(128)(h0, h1, idx, zeros128)
    h0, h1 = _make_layer(128)(h0, h1, a0, a1, deg16, w2l, b2, w2r)

    a0, a1 = _make_seg_sum(128)(h0, h1, idx, zeros128)
    h0, h1 = _make_layer(128)(h0, h1, a0, a1, deg16, w3l, b3, w3r)

    out = _make_pool_mlp()(h0, h1, cnt64, l1w, l1b, l2w, l2b)
    return out[:, :10]


# speedup vs baseline: 3.2327x; 1.1009x over previous
import functools

import jax
import jax.numpy as jnp
from jax import lax
from jax.experimental import pallas as pl
from jax.experimental.pallas import tpu as pltpu
from jax.experimental.pallas import tpu_sc as plsc

N_NODES = 10000
N_PAD = 10752
B_PAD = 10240
N_EDGES = 320000
E_PAD = 327680
NSUB = 16
N_CHUNKS = 160
CHUNK = 128
IBLK = 16
STRIPE = N_PAD // NSUB
NUM_GRAPHS = 64
MAX_NODES = 512
K = 10
HIDDEN = 256
NEG = -1e30


def _sc_mesh():
    return plsc.VectorSubcoreMesh(
        core_axis_name="c", subcore_axis_name="s", num_cores=2, num_subcores=16
    )


@functools.cache
def _make_seg_sum(dh):
    def body(h0_hbm, h1_hbm, idx_hbm, zeros_hbm, out0, out1,
             src_v, dst_v, rows_v, acc, sem):
        cid = lax.axis_index("c")
        sid = lax.axis_index("s")
        pltpu.sync_copy(zeros_hbm.at[pl.ds(sid * STRIPE, STRIPE)],
                        acc.at[pl.ds(sid * STRIPE, STRIPE)])
        plsc.subcore_barrier()

        def run(table):
            def blk(b, carry):
                pltpu.sync_copy(idx_hbm.at[0, sid, pl.ds(b * IBLK, IBLK)],
                                src_v)
                pltpu.sync_copy(idx_hbm.at[1, sid, pl.ds(b * IBLK, IBLK)],
                                dst_v)
                pltpu.async_copy(table.at[src_v.at[0]], rows_v.at[0], sem)

                def step(j, carry2):
                    pltpu.make_async_copy(table.at[src_v.at[j]],
                                          rows_v.at[j & 1], sem).wait()

                    @pl.when(j < IBLK - 1)
                    def _():
                        pltpu.async_copy(table.at[src_v.at[j + 1]],
                                         rows_v.at[(j + 1) & 1], sem)

                    pltpu.sync_copy(rows_v.at[j & 1], acc.at[dst_v.at[j]],
                                    add=True)
                    return carry2
                lax.fori_loop(0, IBLK, step, 0)
                return carry
            lax.fori_loop(0, N_CHUNKS // IBLK, blk, 0)

        @pl.when(cid == 0)
        def _():
            run(h0_hbm)

        @pl.when(cid == 1)
        def _():
            run(h1_hbm)

        plsc.subcore_barrier()

        @pl.when(cid == 0)
        def _():
            pltpu.sync_copy(acc.at[pl.ds(sid * STRIPE, STRIPE)],
                            out0.at[pl.ds(sid * STRIPE, STRIPE)])

        @pl.when(cid == 1)
        def _():
            pltpu.sync_copy(acc.at[pl.ds(sid * STRIPE, STRIPE)],
                            out1.at[pl.ds(sid * STRIPE, STRIPE)])

    return pl.kernel(
        body,
        out_type=[jax.ShapeDtypeStruct((N_PAD, dh), jnp.float32)] * 2,
        mesh=_sc_mesh(),
        scratch_types=[
            pltpu.VMEM((IBLK, CHUNK), jnp.int32),
            pltpu.VMEM((IBLK, CHUNK), jnp.int32),
            pltpu.VMEM((2, CHUNK, dh), jnp.float32),
            pltpu.VMEM_SHARED((N_PAD, dh), jnp.float32),
            pltpu.SemaphoreType.DMA,
        ],
    )


@functools.cache
def _make_seg_partial():
    dh = 128
    half_chunks = N_CHUNKS // 2

    def body(h_hbm, idx_hbm, zeros_hbm, out0, out1,
             src_v, dst_v, rows_v, acc, sem):
        cid = lax.axis_index("c")
        sid = lax.axis_index("s")
        base = cid * half_chunks
        pltpu.sync_copy(zeros_hbm.at[pl.ds(sid * STRIPE, STRIPE)],
                        acc.at[pl.ds(sid * STRIPE, STRIPE)])
        plsc.subcore_barrier()

        def blk(b, carry):
            pltpu.sync_copy(idx_hbm.at[0, sid, pl.ds(base + b * IBLK, IBLK)],
                            src_v)
            pltpu.sync_copy(idx_hbm.at[1, sid, pl.ds(base + b * IBLK, IBLK)],
                            dst_v)
            pltpu.async_copy(h_hbm.at[src_v.at[0]], rows_v.at[0], sem)

            def step(j, carry2):
                pltpu.make_async_copy(h_hbm.at[src_v.at[j]],
                                      rows_v.at[j & 1], sem).wait()

                @pl.when(j < IBLK - 1)
                def _():
                    pltpu.async_copy(h_hbm.at[src_v.at[j + 1]],
                                     rows_v.at[(j + 1) & 1], sem)

                pltpu.sync_copy(rows_v.at[j & 1], acc.at[dst_v.at[j]],
                                add=True)
                return carry2
            lax.fori_loop(0, IBLK, step, 0)
            return carry
        lax.fori_loop(0, half_chunks // IBLK, blk, 0)

        plsc.subcore_barrier()

        @pl.when(cid == 0)
        def _():
            pltpu.sync_copy(acc.at[pl.ds(sid * STRIPE, STRIPE)],
                            out0.at[pl.ds(sid * STRIPE, STRIPE)])

        @pl.when(cid == 1)
        def _():
            pltpu.sync_copy(acc.at[pl.ds(sid * STRIPE, STRIPE)],
                            out1.at[pl.ds(sid * STRIPE, STRIPE)])

    return pl.kernel(
        body,
        out_type=[jax.ShapeDtypeStruct((N_PAD, dh), jnp.float32)] * 2,
        mesh=_sc_mesh(),
        scratch_types=[
            pltpu.VMEM((IBLK, CHUNK), jnp.int32),
            pltpu.VMEM((IBLK, CHUNK), jnp.int32),
            pltpu.VMEM((2, CHUNK, dh), jnp.float32),
            pltpu.VMEM_SHARED((N_PAD, dh), jnp.float32),
            pltpu.SemaphoreType.DMA,
        ],
    )


@functools.cache
def _make_deg_cnt():
    def body(dst_hbm, batch_hbm, ones_hbm, zeros_hbm, deg_out, cnt_out,
             idx_v, ones_v, acc, sem):
        cid = lax.axis_index("c")
        sid = lax.axis_index("s")
        pltpu.sync_copy(ones_hbm, ones_v)
        pltpu.sync_copy(zeros_hbm.at[pl.ds(sid * STRIPE, STRIPE)],
                        acc.at[pl.ds(sid * STRIPE, STRIPE)])
        plsc.subcore_barrier()

        @pl.when(cid == 0)
        def _():
            def blk(b, carry):
                pltpu.sync_copy(dst_hbm.at[sid, pl.ds(b * IBLK, IBLK)],
                                idx_v)

                def step(j, carry2):
                    pltpu.sync_copy(ones_v, acc.at[idx_v.at[j]], add=True)
                    return carry2
                lax.fori_loop(0, IBLK, step, 0)
                return carry
            lax.fori_loop(0, N_CHUNKS // IBLK, blk, 0)

        @pl.when(cid == 1)
        def _():
            pltpu.sync_copy(batch_hbm.at[sid], idx_v.at[pl.ds(0, 5)])

            def step(j, carry):
                pltpu.sync_copy(ones_v, acc.at[idx_v.at[j]], add=True)
                return carry
            lax.fori_loop(0, 5, step, 0)

        plsc.subcore_barrier()

        @pl.when(cid == 0)
        def _():
            pltpu.sync_copy(acc.at[pl.ds(sid * STRIPE, STRIPE)],
                            deg_out.at[pl.ds(sid * STRIPE, STRIPE)])

        @pl.when(jnp.logical_and(cid == 1, sid == 0))
        def _():
            pltpu.sync_copy(acc.at[pl.ds(0, CHUNK)], cnt_out)

    return pl.kernel(
        body,
        out_type=[jax.ShapeDtypeStruct((N_PAD, 128), jnp.float32),
                  jax.ShapeDtypeStruct((CHUNK, 128), jnp.float32)],
        mesh=_sc_mesh(),
        scratch_types=[
            pltpu.VMEM((IBLK, CHUNK), jnp.int32),
            pltpu.VMEM((CHUNK, 128), jnp.float32),
            pltpu.VMEM_SHARED((N_PAD, 128), jnp.float32),
            pltpu.SemaphoreType.DMA,
        ],
    )


BR = 512


@functools.cache
def _make_layer1():
    din = 128

    def body(h, p0, p1, deg, wl, bl, wr, o0, o1):
        agg = p0[...] + p1[...]
        rdeg = 1.0 / jnp.maximum(deg[...][:, :1], 1.0)
        y = (jnp.dot(agg * rdeg, wl[...], preferred_element_type=jnp.float32)
             + bl[...]
             + jnp.dot(h[...], wr[...], preferred_element_type=jnp.float32))
        y = jnp.maximum(y, 0.0)
        o0[...] = y[:, :HIDDEN // 2]
        o1[...] = y[:, HIDDEN // 2:]

    grid = N_PAD // BR
    blk = lambda c: pl.BlockSpec((BR, c), lambda i: (i, 0))
    full = lambda r, c: pl.BlockSpec((r, c), lambda i: (0, 0))
    return pl.pallas_call(
        body,
        grid=(grid,),
        in_specs=[blk(din), blk(din), blk(din),
                  pl.BlockSpec((BR, 128), lambda i: (i, 0)),
                  full(din, HIDDEN), full(1, HIDDEN), full(din, HIDDEN)],
        out_specs=[pl.BlockSpec((BR, HIDDEN // 2), lambda i: (i, 0))] * 2,
        out_shape=[jax.ShapeDtypeStruct((N_PAD, HIDDEN // 2), jnp.float32)] * 2,
    )


@functools.cache
def _make_layer(dh_in):
    din = 2 * dh_in

    def body(h0, h1, a0, a1, deg, wl, bl, wr, o0, o1):
        h = jnp.concatenate([h0[...], h1[...]], axis=1)
        agg = jnp.concatenate([a0[...], a1[...]], axis=1)
        rdeg = 1.0 / jnp.maximum(deg[...][:, :1], 1.0)
        y = (jnp.dot(agg * rdeg, wl[...], preferred_element_type=jnp.float32)
             + bl[...]
             + jnp.dot(h, wr[...], preferred_element_type=jnp.float32))
        y = jnp.maximum(y, 0.0)
        o0[...] = y[:, :HIDDEN // 2]
        o1[...] = y[:, HIDDEN // 2:]

    grid = N_PAD // BR
    half = pl.BlockSpec((BR, dh_in), lambda i: (i, 0))
    full = lambda r, c: pl.BlockSpec((r, c), lambda i: (0, 0))
    return pl.pallas_call(
        body,
        grid=(grid,),
        in_specs=[half, half, half, half,
                  pl.BlockSpec((BR, 128), lambda i: (i, 0)),
                  full(din, HIDDEN), full(1, HIDDEN), full(din, HIDDEN)],
        out_specs=[pl.BlockSpec((BR, HIDDEN // 2), lambda i: (i, 0))] * 2,
        out_shape=[jax.ShapeDtypeStruct((N_PAD, HIDDEN // 2), jnp.float32)] * 2,
    )


@functools.cache
def _make_pool_mlp():
    hh = HIDDEN // 2

    def body(h0_ref, h1_ref, cnt_ref, l1w_ref, l1b_ref, l2w_ref, l2b_ref,
             out_ref, keys_ref, pooled_ref):
        counts = cnt_ref[...][:, :1]
        r64 = lax.broadcasted_iota(jnp.int32, (NUM_GRAPHS, NUM_GRAPHS), 0)
        c64 = lax.broadcasted_iota(jnp.int32, (NUM_GRAPHS, NUM_GRAPHS), 1)
        tril = jnp.where(c64 < r64, 1.0, 0.0)
        starts = jnp.dot(tril, counts,
                         preferred_element_type=jnp.float32).astype(jnp.int32)
        cnt_i = counts.astype(jnp.int32)

        rio_col = lax.broadcasted_iota(jnp.int32, (MAX_NODES, 1), 0)
        for g in range(NUM_GRAPHS):
            s = starts[g, 0]
            krow = h1_ref[pl.ds(s, MAX_NODES), pl.ds(hh - 1, 1)]
            krow = jnp.where(rio_col < cnt_i[g, 0], krow, NEG)
            keys_ref[:, pl.ds(g, 1)] = krow

        keys = keys_ref[...]
        rio = lax.broadcasted_iota(jnp.int32, (MAX_NODES, NUM_GRAPHS), 0)
        for k in range(K):
            m = jnp.max(keys, axis=0, keepdims=True)
            cand = jnp.where(keys == m, rio, MAX_NODES * 2)
            idx = jnp.min(cand, axis=0, keepdims=True)
            for g in range(NUM_GRAPHS):
                s = starts[g, 0] + idx[0, g]
                row = jnp.concatenate(
                    [h0_ref[pl.ds(s, 1), :], h1_ref[pl.ds(s, 1), :]], axis=1)
                row = jnp.where(m[0, g] > (NEG * 0.5), row, 0.0)
                pooled_ref[pl.ds(g, 1), pl.ds(k * HIDDEN, HIDDEN)] = row
            keys = jnp.where(rio == idx, NEG, keys)

        pooled = pooled_ref[...]
        z = jnp.dot(pooled, l1w_ref[...],
                    preferred_element_type=jnp.float32) + l1b_ref[...]
        z = jnp.maximum(z, 0.0)
        logits = jnp.dot(z, l2w_ref[...],
                         preferred_element_type=jnp.float32) + l2b_ref[...]
        cmask = lax.broadcasted_iota(jnp.int32, logits.shape, 1) < 10
        lmax = jnp.max(jnp.where(cmask, logits, NEG), axis=1, keepdims=True)
        ex = jnp.where(cmask, jnp.exp(logits - lmax), 0.0)
        lse = jnp.log(jnp.sum(ex, axis=1, keepdims=True))
        out_ref[...] = logits - lmax - lse

    return pl.pallas_call(
        body,
        out_shape=jax.ShapeDtypeStruct((NUM_GRAPHS, 128), jnp.float32),
        scratch_shapes=[
            pltpu.VMEM((MAX_NODES, NUM_GRAPHS), jnp.float32),
            pltpu.VMEM((NUM_GRAPHS, K * HIDDEN), jnp.float32),
        ],
    )


def kernel(x, edge_index, batch,
           conv1_Wl, conv1_bl, conv1_Wr,
           conv2_Wl, conv2_bl, conv2_Wr,
           conv3_Wl, conv3_bl, conv3_Wr,
           lin1_W, lin1_b, lin2_W, lin2_b):
    f32 = jnp.float32
    x_pad = jnp.pad(x, ((0, N_PAD - N_NODES), (0, 0)))
    pad_e = E_PAD - N_EDGES
    dump = jnp.full((pad_e,), N_PAD - 1, jnp.int32)
    src_p = jnp.concatenate([edge_index[0], dump])
    dst_p = jnp.concatenate([edge_index[1], dump])
    idx = jnp.stack([src_p, dst_p]).reshape(2, NSUB, N_CHUNKS, CHUNK)
    dst_rs = dst_p.reshape(NSUB, N_CHUNKS, CHUNK)
    batch_rs = jnp.concatenate(
        [batch, jnp.full((B_PAD - N_NODES,), NUM_GRAPHS, jnp.int32)]
    ).reshape(NSUB, 5, CHUNK)
    ones128 = jnp.ones((CHUNK, 128), f32)
    zeros128 = jnp.zeros((N_PAD, 128), f32)

    w1l, w1r = conv1_Wl.T, conv1_Wr.T
    w2l, w2r = conv2_Wl.T, conv2_Wr.T
    w3l, w3r = conv3_Wl.T, conv3_Wr.T
    b1 = conv1_bl.reshape(1, HIDDEN)
    b2 = conv2_bl.reshape(1, HIDDEN)
    b3 = conv3_bl.reshape(1, HIDDEN)
    l1w = lin1_W.T
    l1b = lin1_b.reshape(1, HIDDEN)
    l2w = jnp.pad(lin2_W.T, ((0, 0), (0, 128 - lin2_W.shape[0])))
    l2b = jnp.pad(lin2_b, (0, 128 - lin2_b.shape[0])).reshape(1, 128)

    deg16, cnt128 = _make_deg_cnt()(dst_rs, batch_rs, ones128, zeros128)
    cnt64 = cnt128[:NUM_GRAPHS]

    p0, p1 = _make_seg_partial()(x_pad, idx, zeros128)
    h0, h1 = _make_layer1()(x_pad, p0, p1, deg16, w1l, b1, w1r)

    a0, a1 = _make_seg_sum(128)(h0, h1, idx, zeros128)
    h0, h1 = _make_layer(128)(h0, h1, a0, a1, deg16, w2l, b2, w2r)

    a0, a1 = _make_seg_sum(128)(h0, h1, idx, zeros128)
    h0, h1 = _make_layer(128)(h0, h1, a0, a1, deg16, w3l, b3, w3r)

    out = _make_pool_mlp()(h0, h1, cnt64, l1w, l1b, l2w, l2b)
    return out[:, :10]
